# baseline (speedup 1.0000x reference)
import math

import jax
import jax.numpy as jnp
from jax import lax
from jax.experimental import pallas as pl
from jax.experimental.pallas import tpu as pltpu
from jax.experimental.pallas import tpu_sc as plsc

N_TOKEN = 1000000
D_MODEL = 64
SCALE = math.sqrt(D_MODEL)

NC, NS = 2, 16
NW = NC * NS
N_BATCH = 4096
N_TOK = 200
ROWS_PER_W = N_BATCH // NW
CHUNK = 100
SPLIT = N_TOK // CHUNK
N_CHUNKS = ROWS_PER_W * SPLIT
NBUF = 4
LOOKAHEAD = 2


def _emb_kernel(x_hbm, lut_hbm, out_hbm, idx_v, rows, *sems):
    g_sems = sems[:NBUF]
    s_sems = sems[NBUF:]
    wid = lax.axis_index("s") * NC + lax.axis_index("c")
    row0 = wid * ROWS_PER_W
    pltpu.sync_copy(x_hbm.at[wid], idx_v)

    def out_slice(j):
        return out_hbm.at[row0 + j // SPLIT, pl.ds((j % SPLIT) * CHUNK, CHUNK)]

    def gather_start(j, b):
        pltpu.async_copy(lut_hbm.at[idx_v.at[j]], rows.at[b], g_sems[b])

    def gather_wait(j, b):
        pltpu.make_async_copy(lut_hbm.at[idx_v.at[j]], rows.at[b],
                              g_sems[b]).wait()

    def store_start(j, b):
        pltpu.async_copy(rows.at[b], out_slice(j), s_sems[b])

    def store_wait(b):
        pltpu.make_async_copy(rows.at[b], out_slice(0), s_sems[b]).wait()

    def scale(b):
        @pl.loop(0, CHUNK, unroll=4)
        def _row(r):
            for c in range(D_MODEL // 16):
                sl = pl.ds(c * 16, 16)
                rows[b, r, sl] = rows[b, r, sl] * SCALE

    def process(j, b, issue_j=None, issue_wait=True):
        if issue_j is not None:
            bb = (b + LOOKAHEAD) % NBUF
            if issue_wait:
                store_wait(bb)
            gather_start(issue_j, bb)
        gather_wait(j, b)
        scale(b)
        store_start(j, b)

    for j in range(LOOKAHEAD):
        gather_start(j, j % NBUF)

    for b in range(NBUF):
        process(b, b, issue_j=b + LOOKAHEAD,
                issue_wait=b + LOOKAHEAD >= NBUF)

    @pl.loop(NBUF, N_CHUNKS - NBUF, step=NBUF)
    def _grp(j0):
        for b in range(NBUF):
            process(j0 + b, b, issue_j=j0 + b + LOOKAHEAD)

    for b in range(NBUF):
        j = N_CHUNKS - NBUF + b
        process(j, b,
                issue_j=(j + LOOKAHEAD) if j + LOOKAHEAD < N_CHUNKS else None)

    for b in range(NBUF):
        store_wait(b)


@jax.jit
def _emb(x3, lut):
    mesh = plsc.VectorSubcoreMesh(core_axis_name="c", subcore_axis_name="s")
    f = pl.kernel(
        _emb_kernel,
        out_type=jax.ShapeDtypeStruct((N_BATCH, N_TOK, D_MODEL), jnp.float32),
        mesh=mesh,
        compiler_params=pltpu.CompilerParams(use_tc_tiling_on_sc=False),
        scratch_types=(
            [pltpu.VMEM((N_CHUNKS, CHUNK), jnp.int32),
             pltpu.VMEM((NBUF, CHUNK, D_MODEL), jnp.float32)]
            + [pltpu.SemaphoreType.DMA] * (2 * NBUF)
        ),
    )
    return f(x3, lut)


def kernel(x, lut):
    x3 = x.reshape(NW, N_CHUNKS, CHUNK).astype(jnp.int32)
    return _emb(x3, lut)
